# baseline (speedup 1.0000x reference)
import functools

import jax
import jax.numpy as jnp
from jax import lax
from jax.experimental import pallas as pl
from jax.experimental.pallas import tpu as pltpu
from jax.experimental.pallas import tpu_sc as plsc

_NC = 2
_NS = 16
_NW = _NC * _NS

_N_CATS = 26
_EMB = 64
_NQ = 3
_ROW = _EMB + _NQ
_OUT_W = 4 * _ROW
_BLK = 128
_GRP = _BLK // 16


@functools.lru_cache(maxsize=None)
def _make_lookup(batch):
    rows_per_w = batch // _NW
    n_blocks = rows_per_w // _BLK
    mesh = plsc.VectorSubcoreMesh(core_axis_name="c", subcore_axis_name="s")

    @functools.partial(
        pl.kernel,
        out_type=jax.ShapeDtypeStruct((_OUT_W, batch), jnp.float32),
        mesh=mesh,
        compiler_params=pltpu.CompilerParams(needs_layout_passes=False),
        scratch_types=[
            pltpu.VMEM((4, _N_CATS, _EMB), jnp.float32),
            pltpu.VMEM((4, _N_CATS, _NQ), jnp.float32),
            pltpu.VMEM((4, rows_per_w), jnp.int32),
            pltpu.VMEM((2, _OUT_W, _BLK), jnp.float32),
            pltpu.SemaphoreType.DMA,
            pltpu.SemaphoreType.DMA,
            pltpu.SemaphoreType.DMA,
        ],
    )
    def lookup_kernel(ca, cb, cc, cd, ea, eb, ec, ed, qa, qb, qc, qd,
                      out_hbm, emb_v, quant_v, idx_v, stag_v, ldsem,
                      sem0, sem1):
        wid = lax.axis_index("s") * _NC + lax.axis_index("c")
        base = wid * rows_per_w
        loads = []
        for f, (e_hbm, q_hbm, c_hbm) in enumerate(
                zip((ea, eb, ec, ed), (qa, qb, qc, qd), (ca, cb, cc, cd))):
            loads.append(pltpu.async_copy(e_hbm, emb_v.at[f], ldsem))
            loads.append(pltpu.async_copy(q_hbm, quant_v.at[f], ldsem))
            loads.append(pltpu.async_copy(
                c_hbm.at[pl.ds(base, rows_per_w)], idx_v.at[f], ldsem))
        for ld in loads:
            ld.wait()

        sems = [sem0, sem1]
        pending = [None, None]
        for q in range(n_blocks):
            p = q % 2
            if pending[p] is not None:
                pending[p].wait()

            def grp(g, carry, q=q, p=p):
                gidx = q * _GRP + g
                vrow = [idx_v[f, pl.ds(gidx * 16, 16)] for f in range(4)]

                @plsc.parallel_loop(0, _EMB, unroll=8)
                def colstep(c):
                    cv = jnp.full((16,), 0, jnp.int32) + c
                    for f in range(4):
                        vals = plsc.load_gather(emb_v.at[f], [vrow[f], cv])
                        stag_v.at[p][f * _ROW + c, pl.ds(g * 16, 16)] = vals

                for jq in range(_NQ):
                    cq = jnp.full((16,), jq, jnp.int32)
                    for f in range(4):
                        vals = plsc.load_gather(quant_v.at[f], [vrow[f], cq])
                        stag_v.at[p][f * _ROW + _EMB + jq,
                                     pl.ds(g * 16, 16)] = vals
                return carry

            lax.fori_loop(0, _GRP, grp, 0)
            pending[p] = pltpu.async_copy(
                stag_v.at[p],
                out_hbm.at[:, pl.ds(base + q * _BLK, _BLK)], sems[p])
        for p in range(2):
            if pending[p] is not None:
                pending[p].wait()

    return lookup_kernel


def kernel(cat_a, cat_b, cat_c, cat_d,
           emb_cat_a, emb_cat_b, emb_cat_c, emb_cat_d,
           quant_cat_a, quant_cat_b, quant_cat_c, quant_cat_d):
    batch = cat_a.shape[0]
    out_t = _make_lookup(batch)(
        cat_a, cat_b, cat_c, cat_d,
        emb_cat_a, emb_cat_b, emb_cat_c, emb_cat_d,
        quant_cat_a, quant_cat_b, quant_cat_c, quant_cat_d)
    return out_t.T


# speedup vs baseline: 2.1768x; 2.1768x over previous
import functools

import jax
import jax.numpy as jnp
from jax import lax
from jax.experimental import pallas as pl
from jax.experimental.pallas import tpu as pltpu
from jax.experimental.pallas import tpu_sc as plsc

_NC = 2
_NS = 16
_NW = _NC * _NS

_N_CATS = 26
_EMB = 64
_NQ = 3
_ROW = _EMB + _NQ
_OUT_W = 4 * _ROW
_BLK = 128
_GRP = _BLK // 16


@functools.lru_cache(maxsize=None)
def _make_lookup(batch):
    rows_per_w = batch // _NW
    n_blocks = rows_per_w // _BLK
    mesh = plsc.VectorSubcoreMesh(core_axis_name="c", subcore_axis_name="s")

    @functools.partial(
        pl.kernel,
        out_type=jax.ShapeDtypeStruct((_OUT_W, batch), jnp.float32),
        mesh=mesh,
        compiler_params=pltpu.CompilerParams(needs_layout_passes=False),
        scratch_types=[
            pltpu.VMEM((4, _N_CATS, _EMB), jnp.float32),
            pltpu.VMEM((4, _N_CATS, _NQ), jnp.float32),
            pltpu.VMEM((4 * _N_CATS * _ROW,), jnp.float32),
            pltpu.VMEM((4, rows_per_w), jnp.int32),
            pltpu.VMEM((2, _OUT_W, _BLK), jnp.float32),
            pltpu.SemaphoreType.DMA,
            pltpu.SemaphoreType.DMA,
            pltpu.SemaphoreType.DMA,
        ],
    )
    def lookup_kernel(ca, cb, cc, cd, ea, eb, ec, ed, qa, qb, qc, qd,
                      out_hbm, emb_v, quant_v, tab_v, idx_v, stag_v, ldsem,
                      sem0, sem1):
        wid = lax.axis_index("s") * _NC + lax.axis_index("c")
        base = wid * rows_per_w
        loads = []
        for f, (e_hbm, q_hbm, c_hbm) in enumerate(
                zip((ea, eb, ec, ed), (qa, qb, qc, qd), (ca, cb, cc, cd))):
            loads.append(pltpu.async_copy(e_hbm, emb_v.at[f], ldsem))
            loads.append(pltpu.async_copy(q_hbm, quant_v.at[f], ldsem))
            loads.append(pltpu.async_copy(
                c_hbm.at[pl.ds(base, rows_per_w)], idx_v.at[f], ldsem))
        for ld in loads:
            ld.wait()

        lane = lax.iota(jnp.int32, 16)
        qmask = lane < _NQ
        for f in range(4):
            def repack(r, carry, f=f):
                rbase = jnp.full((16,), 0, jnp.int32) + (f * _N_CATS + r) * _ROW
                for k in range(_EMB // 16):
                    vals = emb_v.at[f][r, pl.ds(k * 16, 16)]
                    plsc.store_scatter(tab_v, [rbase + k * 16 + lane], vals)
                qv = plsc.load_gather(quant_v.at[f],
                                      [jnp.full((16,), 0, jnp.int32) + r,
                                       lane], mask=qmask)
                plsc.store_scatter(tab_v, [rbase + _EMB + lane], qv,
                                   mask=qmask)
                return carry
            lax.fori_loop(0, _N_CATS, repack, 0)

        sems = [sem0, sem1]
        pending = [None, None]
        for q in range(n_blocks):
            p = q % 2
            if pending[p] is not None:
                pending[p].wait()

            def grp(g, carry, q=q, p=p):
                gidx = q * _GRP + g
                vf = [(idx_v[f, pl.ds(gidx * 16, 16)] + f * _N_CATS) * _ROW
                      for f in range(4)]

                @plsc.parallel_loop(0, _ROW, unroll=8)
                def colstep(c):
                    for f in range(4):
                        vals = plsc.load_gather(tab_v, [vf[f] + c])
                        stag_v.at[p][f * _ROW + c, pl.ds(g * 16, 16)] = vals

                return carry

            lax.fori_loop(0, _GRP, grp, 0)
            pending[p] = pltpu.async_copy(
                stag_v.at[p],
                out_hbm.at[:, pl.ds(base + q * _BLK, _BLK)], sems[p])
        for p in range(2):
            if pending[p] is not None:
                pending[p].wait()

    return lookup_kernel


def kernel(cat_a, cat_b, cat_c, cat_d,
           emb_cat_a, emb_cat_b, emb_cat_c, emb_cat_d,
           quant_cat_a, quant_cat_b, quant_cat_c, quant_cat_d):
    batch = cat_a.shape[0]
    out_t = _make_lookup(batch)(
        cat_a, cat_b, cat_c, cat_d,
        emb_cat_a, emb_cat_b, emb_cat_c, emb_cat_d,
        quant_cat_a, quant_cat_b, quant_cat_c, quant_cat_d)
    return out_t.T


# speedup vs baseline: 2.3361x; 1.0732x over previous
import functools

import jax
import jax.numpy as jnp
from jax import lax
from jax.experimental import pallas as pl
from jax.experimental.pallas import tpu as pltpu
from jax.experimental.pallas import tpu_sc as plsc

_NC = 2
_NS = 16
_NW = _NC * _NS

_N_CATS = 26
_ROW = 64 + 3
_OUT_W = 4 * _ROW
_BLK = 128
_GRP = _BLK // 16


@functools.lru_cache(maxsize=None)
def _make_lookup(batch):
    rows_per_w = batch // _NW
    n_blocks = rows_per_w // _BLK
    mesh = plsc.VectorSubcoreMesh(core_axis_name="c", subcore_axis_name="s")

    @functools.partial(
        pl.kernel,
        out_type=jax.ShapeDtypeStruct((_OUT_W, batch), jnp.float32),
        mesh=mesh,
        compiler_params=pltpu.CompilerParams(needs_layout_passes=False,
                                             skip_device_barrier=True),
        scratch_types=[
            pltpu.VMEM((4 * _N_CATS * _ROW,), jnp.float32),
            pltpu.VMEM((4, rows_per_w), jnp.int32),
            pltpu.VMEM((2, _OUT_W, _BLK), jnp.float32),
            pltpu.SemaphoreType.DMA,
            pltpu.SemaphoreType.DMA,
            pltpu.SemaphoreType.DMA,
        ],
    )
    def lookup_kernel(tab_hbm, idx_hbm, out_hbm, tab_v, idx_v, stag_v,
                      ldsem, sem0, sem1):
        wid = lax.axis_index("s") * _NC + lax.axis_index("c")
        base = wid * rows_per_w
        loads = [pltpu.async_copy(tab_hbm, tab_v, ldsem)]
        for f in range(4):
            loads.append(pltpu.async_copy(
                idx_hbm.at[f, wid], idx_v.at[f], ldsem))
        for ld in loads:
            ld.wait()

        sems = [sem0, sem1]
        pending = [None, None]
        for q in range(n_blocks):
            p = q % 2
            if pending[p] is not None:
                pending[p].wait()

            def grp(g, carry, q=q, p=p):
                gidx = q * _GRP + g
                vf = [idx_v[f, pl.ds(gidx * 16, 16)] * _ROW
                      + f * _N_CATS * _ROW for f in range(4)]

                @plsc.parallel_loop(0, _ROW, unroll=8)
                def colstep(c):
                    for f in range(4):
                        vals = plsc.load_gather(tab_v, [vf[f] + c])
                        stag_v.at[p][f * _ROW + c, pl.ds(g * 16, 16)] = vals

                return carry

            lax.fori_loop(0, _GRP, grp, 0)
            pending[p] = pltpu.async_copy(
                stag_v.at[p],
                out_hbm.at[:, pl.ds(base + q * _BLK, _BLK)], sems[p])
        for p in range(2):
            if pending[p] is not None:
                pending[p].wait()

    return lookup_kernel


def kernel(cat_a, cat_b, cat_c, cat_d,
           emb_cat_a, emb_cat_b, emb_cat_c, emb_cat_d,
           quant_cat_a, quant_cat_b, quant_cat_c, quant_cat_d):
    table = jnp.concatenate([
        jnp.concatenate([emb_cat_a, quant_cat_a], axis=1),
        jnp.concatenate([emb_cat_b, quant_cat_b], axis=1),
        jnp.concatenate([emb_cat_c, quant_cat_c], axis=1),
        jnp.concatenate([emb_cat_d, quant_cat_d], axis=1),
    ], axis=0).reshape(-1)

    batch = cat_a.shape[0]
    rows_per_w = batch // _NW
    idx3 = jnp.stack([cat_a, cat_b, cat_c, cat_d],
                     axis=0).reshape(4, _NW, rows_per_w)

    out_t = _make_lookup(batch)(table, idx3)
    return out_t.T


# speedup vs baseline: 2.3706x; 1.0148x over previous
import functools

import jax
import jax.numpy as jnp
from jax import lax
from jax.experimental import pallas as pl
from jax.experimental.pallas import tpu as pltpu
from jax.experimental.pallas import tpu_sc as plsc

_NC = 2
_NS = 16
_NW = _NC * _NS

_N_CATS = 26
_EMB = 64
_NQ = 3
_ROW = _EMB + _NQ
_OUT_W = 4 * _ROW
_BLK = 128
_GRP = _BLK // 16


@functools.lru_cache(maxsize=None)
def _make_lookup(batch):
    rows_per_w = batch // _NW
    n_blocks = rows_per_w // _BLK
    mesh = plsc.VectorSubcoreMesh(core_axis_name="c", subcore_axis_name="s")

    @functools.partial(
        pl.kernel,
        out_type=jax.ShapeDtypeStruct((_OUT_W, batch), jnp.float32),
        mesh=mesh,
        compiler_params=pltpu.CompilerParams(needs_layout_passes=False,
                                             skip_device_barrier=True),
        scratch_types=[
            pltpu.VMEM((4, _N_CATS, _EMB), jnp.float32),
            pltpu.VMEM((4, _NQ, _N_CATS), jnp.float32),
            pltpu.VMEM((4 * _N_CATS * _ROW,), jnp.float32),
            pltpu.VMEM((4, rows_per_w), jnp.int32),
            pltpu.VMEM((2, _OUT_W, _BLK), jnp.float32),
            pltpu.SemaphoreType.DMA,
            pltpu.SemaphoreType.DMA,
            pltpu.SemaphoreType.DMA,
        ],
    )
    def lookup_kernel(idx_hbm, ea, eb, ec, ed, qa, qb, qc, qd,
                      out_hbm, emb_v, qt_v, tab_v, idx_v, stag_v,
                      ldsem, sem0, sem1):
        wid = lax.axis_index("s") * _NC + lax.axis_index("c")
        base = wid * rows_per_w
        loads = []
        for f, (e_hbm, qt_hbm) in enumerate(
                zip((ea, eb, ec, ed), (qa, qb, qc, qd))):
            loads.append(pltpu.async_copy(e_hbm, emb_v.at[f], ldsem))
            loads.append(pltpu.async_copy(qt_hbm, qt_v.at[f], ldsem))
            loads.append(pltpu.async_copy(
                idx_hbm.at[f, wid], idx_v.at[f], ldsem))
        for ld in loads:
            ld.wait()

        lane = lax.iota(jnp.int32, 16)
        lane67 = lane * _ROW
        zero = jnp.full((16,), 0, jnp.int32)

        for f in range(4):
            def repack(r, carry, f=f):
                rbase = zero + (f * _N_CATS + r) * _ROW
                for k in range(_EMB // 16):
                    vals = emb_v.at[f][r, pl.ds(k * 16, 16)]
                    plsc.store_scatter(tab_v, [rbase + (k * 16) + lane], vals)
                return carry
            lax.fori_loop(0, _N_CATS, repack, 0)
            for c in range(_NQ):
                for r0 in (0, _N_CATS - 16):
                    vals = qt_v.at[f][c, pl.ds(r0, 16)]
                    plsc.store_scatter(
                        tab_v,
                        [zero + (f * _N_CATS + r0) * _ROW + _EMB + c + lane67],
                        vals)

        sems = [sem0, sem1]
        pending = [None, None]
        for q in range(n_blocks):
            p = q % 2
            if pending[p] is not None:
                pending[p].wait()

            def grp(g, carry, q=q, p=p):
                gidx = q * _GRP + g
                vf = [idx_v[f, pl.ds(gidx * 16, 16)] * _ROW
                      + f * _N_CATS * _ROW for f in range(4)]

                @plsc.parallel_loop(0, _ROW, unroll=8)
                def colstep(c):
                    for f in range(4):
                        vals = plsc.load_gather(tab_v, [vf[f] + c])
                        stag_v.at[p][f * _ROW + c, pl.ds(g * 16, 16)] = vals

                return carry

            lax.fori_loop(0, _GRP, grp, 0)
            pending[p] = pltpu.async_copy(
                stag_v.at[p],
                out_hbm.at[:, pl.ds(base + q * _BLK, _BLK)], sems[p])
        for p in range(2):
            if pending[p] is not None:
                pending[p].wait()

    return lookup_kernel


def kernel(cat_a, cat_b, cat_c, cat_d,
           emb_cat_a, emb_cat_b, emb_cat_c, emb_cat_d,
           quant_cat_a, quant_cat_b, quant_cat_c, quant_cat_d):
    batch = cat_a.shape[0]
    rows_per_w = batch // _NW
    idx3 = jnp.concatenate([cat_a, cat_b, cat_c, cat_d]
                           ).reshape(4, _NW, rows_per_w)

    out_t = _make_lookup(batch)(
        idx3, emb_cat_a, emb_cat_b, emb_cat_c, emb_cat_d,
        quant_cat_a.T, quant_cat_b.T, quant_cat_c.T, quant_cat_d.T)
    return out_t.T
